# Initial kernel scaffold; baseline (speedup 1.0000x reference)
import jax
import jax.numpy as jnp
from jax.experimental import pallas as pl


def kernel(x, edge_index, batch, bn_gamma, bn_beta, W1, b1, eps1, W2, b2, eps2, W3, b3, eps3, W4, b4):
    raise NotImplementedError("write your pallas kernel here")


# speedup vs baseline: 3.0170x; 3.0170x over previous
import functools

import jax
import jax.numpy as jnp
import numpy as np
from jax import lax
from jax.experimental import pallas as pl
from jax.experimental.pallas import tpu as pltpu
from jax.experimental.pallas import tpu_sc as plsc

N = 10000
E = 320000
D = 128
HID = 256
OUT = 128

NSUB = 16
CHUNK = 128
EPAD = 327680
NR_E = 80
NR_F = 160
G = 40
NACC = 10112
RPT = NACC // NSUB


def _make_segsum(nrows, edge_split):
  mesh = plsc.VectorSubcoreMesh(core_axis_name="c", subcore_axis_name="s")
  ngroups = nrows // G

  @functools.partial(
      pl.kernel,
      out_type=jax.ShapeDtypeStruct((2, NACC, 128), jnp.float32),
      mesh=mesh,
      scratch_types=[
          pltpu.VMEM((G, CHUNK), jnp.int32),
          pltpu.VMEM((G, CHUNK), jnp.int32),
          pltpu.VMEM((CHUNK, 128), jnp.float32),
          pltpu.VMEM((CHUNK, 128), jnp.float32),
          pltpu.VMEM_SHARED((NACC, 128), jnp.float32),
          pltpu.SemaphoreType.DMA,
          pltpu.SemaphoreType.DMA,
      ],
  )
  def seg(h_hbm, src_hbm, dst_hbm, zeros_hbm, out_hbm,
          src_v, dst_v, buf0, buf1, acc_sh, sem0, sem1):
    c = lax.axis_index("c")
    s = lax.axis_index("s")
    base = s * RPT
    pltpu.sync_copy(zeros_hbm.at[pl.ds(base, RPT)],
                    acc_sh.at[pl.ds(base, RPT)])
    plsc.subcore_barrier()

    for g in range(ngroups):
      pltpu.sync_copy(src_hbm.at[c, s, pl.ds(g * G, G)], src_v)
      if edge_split:
        pltpu.sync_copy(dst_hbm.at[c, s, pl.ds(g * G, G)], dst_v)
      else:
        pltpu.sync_copy(dst_hbm.at[s, pl.ds(g * G, G)], dst_v)

      def body(j2, carry):
        r0 = 2 * j2
        r1 = r0 + 1
        cp0 = pltpu.async_copy(h_hbm.at[src_v.at[r0]], buf0, sem0)
        cp1 = pltpu.async_copy(h_hbm.at[src_v.at[r1]], buf1, sem1)
        cp0.wait()
        pltpu.sync_copy(buf0, acc_sh.at[dst_v.at[r0]], add=True)
        cp1.wait()
        pltpu.sync_copy(buf1, acc_sh.at[dst_v.at[r1]], add=True)
        return carry

      lax.fori_loop(0, G // 2, body, 0)

    plsc.subcore_barrier()
    pltpu.sync_copy(acc_sh.at[pl.ds(base, RPT)],
                    out_hbm.at[c, pl.ds(base, RPT)])

  return seg


_segsum_l1 = _make_segsum(NR_E, edge_split=True)
_segsum_hid = _make_segsum(NR_F, edge_split=False)


def _bn_body(x_ref, sc_ref, sh_ref, out_ref):
  out_ref[...] = x_ref[...] * sc_ref[...] + sh_ref[...]


def _layer1_body(h_ref, agg_ref, w_ref, b_ref, eps_ref, out_ref):
  e = 1.0 + eps_ref[0, 0]
  u = e * h_ref[...] + agg_ref[0, 0:N, :] + agg_ref[1, 0:N, :]
  z = jnp.dot(u, w_ref[...], preferred_element_type=jnp.float32) + b_ref[...]
  z = jnp.maximum(z, 0.0)
  out_ref[0:N] = z[:, 0:128]
  out_ref[N:2 * N] = z[:, 128:256]


def _layer2_body(h_ref, agg_ref, w_ref, b_ref, eps_ref, out_ref):
  e = 1.0 + eps_ref[0, 0]
  u = e * h_ref[0:N, :] + agg_ref[0, 0:N, :]
  v = e * h_ref[N:2 * N, :] + agg_ref[1, 0:N, :]
  z = (jnp.dot(u, w_ref[0:128, :], preferred_element_type=jnp.float32)
       + jnp.dot(v, w_ref[128:256, :], preferred_element_type=jnp.float32)
       + b_ref[...])
  z = jnp.maximum(z, 0.0)
  out_ref[0:N] = z[:, 0:128]
  out_ref[N:2 * N] = z[:, 128:256]


def _layer3_body(h_ref, agg_ref, w3_ref, b3_ref, eps_ref, w4_ref, b4_ref,
                 out_ref):
  e = 1.0 + eps_ref[0, 0]
  u = e * h_ref[0:N, :] + agg_ref[0, 0:N, :]
  v = e * h_ref[N:2 * N, :] + agg_ref[1, 0:N, :]
  t = (jnp.dot(u, w3_ref[0:128, :], preferred_element_type=jnp.float32)
       + jnp.dot(v, w3_ref[128:256, :], preferred_element_type=jnp.float32)
       + b3_ref[...])
  t = jnp.maximum(t, 0.0)
  out_ref[...] = (jnp.dot(t, w4_ref[...], preferred_element_type=jnp.float32)
                  + b4_ref[...])


def _tc_call(body, out_shape, smem_arg_idxs, *args):
  specs = [pl.BlockSpec(memory_space=pltpu.VMEM) for _ in args]
  for i in smem_arg_idxs:
    specs[i] = pl.BlockSpec(memory_space=pltpu.SMEM)
  return pl.pallas_call(
      body,
      out_shape=jax.ShapeDtypeStruct(out_shape, jnp.float32),
      in_specs=specs,
      out_specs=pl.BlockSpec(memory_space=pltpu.VMEM),
  )(*args)


def kernel(x, edge_index, batch, bn_gamma, bn_beta,
           W1, b1, eps1, W2, b2, eps2, W3, b3, eps3, W4, b4):
  del batch
  f32 = jnp.float32
  scale = (bn_gamma * np.float32(1.0 / np.sqrt(1.0 + 1e-5))).reshape(1, D)
  shift = bn_beta.reshape(1, D)

  src = edge_index[0]
  dst = edge_index[1]
  pad = EPAD - E
  srcp = jnp.concatenate([src, jnp.zeros((pad,), jnp.int32)])
  dstp = jnp.concatenate([dst, jnp.full((pad,), N, jnp.int32)])
  src_e = srcp.reshape(2, NSUB, NR_E, CHUNK)
  dst_e = dstp.reshape(2, NSUB, NR_E, CHUNK)
  src_f = jnp.stack([srcp, srcp + N]).reshape(2, NSUB, NR_F, CHUNK)
  dst_f = dstp.reshape(NSUB, NR_F, CHUNK)
  zeros = jnp.zeros((NACC, 128), f32)

  e1 = eps1.astype(f32).reshape(1, 1)
  e2 = eps2.astype(f32).reshape(1, 1)
  e3 = eps3.astype(f32).reshape(1, 1)

  h0 = _tc_call(_bn_body, (N, D), (), x, scale, shift)
  agg1 = _segsum_l1(h0, src_e, dst_e, zeros)
  h1 = _tc_call(_layer1_body, (2 * N, 128), (4,),
                h0, agg1, W1, b1.reshape(1, HID), e1)
  agg2 = _segsum_hid(h1, src_f, dst_f, zeros)
  h2 = _tc_call(_layer2_body, (2 * N, 128), (4,),
                h1, agg2, W2, b2.reshape(1, HID), e2)
  agg3 = _segsum_hid(h2, src_f, dst_f, zeros)
  out = _tc_call(_layer3_body, (N, OUT), (4,), h2, agg3, W3,
                 b3.reshape(1, HID), e3, W4, b4.reshape(1, OUT))
  return out


# speedup vs baseline: 3.0819x; 1.0215x over previous
import functools

import jax
import jax.numpy as jnp
import numpy as np
from jax import lax
from jax.experimental import pallas as pl
from jax.experimental.pallas import tpu as pltpu
from jax.experimental.pallas import tpu_sc as plsc

N = 10000
E = 320000
D = 128
HID = 256
OUT = 128

NSUB = 16
CHUNK = 128
EPAD = 327680
NR_E = 80
NR_F = 160
G = 40
NACC = 10112
RPT = NACC // NSUB


def _make_segsum(nrows, edge_split):
  mesh = plsc.VectorSubcoreMesh(core_axis_name="c", subcore_axis_name="s")
  ngroups = nrows // G

  @functools.partial(
      pl.kernel,
      out_type=jax.ShapeDtypeStruct((2, NACC, 128), jnp.float32),
      mesh=mesh,
      scratch_types=[
          pltpu.VMEM((G, CHUNK), jnp.int32),
          pltpu.VMEM((G, CHUNK), jnp.int32),
          pltpu.VMEM((CHUNK, 128), jnp.float32),
          pltpu.VMEM((CHUNK, 128), jnp.float32),
          pltpu.VMEM_SHARED((NACC, 128), jnp.float32),
          pltpu.SemaphoreType.DMA,
          pltpu.SemaphoreType.DMA,
      ],
  )
  def seg(h_hbm, src_hbm, dst_hbm, zeros_hbm, out_hbm,
          src_v, dst_v, buf0, buf1, acc_sh, sem0, sem1):
    c = lax.axis_index("c")
    s = lax.axis_index("s")
    base = s * RPT
    pltpu.sync_copy(zeros_hbm.at[pl.ds(base, RPT)],
                    acc_sh.at[pl.ds(base, RPT)])
    plsc.subcore_barrier()

    for g in range(ngroups):
      pltpu.sync_copy(src_hbm.at[c, s, pl.ds(g * G, G)], src_v)
      if edge_split:
        pltpu.sync_copy(dst_hbm.at[c, s, pl.ds(g * G, G)], dst_v)
      else:
        pltpu.sync_copy(dst_hbm.at[s, pl.ds(g * G, G)], dst_v)

      def body(j2, carry):
        r0 = 2 * j2
        r1 = r0 + 1
        cp0 = pltpu.async_copy(h_hbm.at[src_v.at[r0]], buf0, sem0)
        cp1 = pltpu.async_copy(h_hbm.at[src_v.at[r1]], buf1, sem1)
        cp0.wait()
        pltpu.sync_copy(buf0, acc_sh.at[dst_v.at[r0]], add=True)
        cp1.wait()
        pltpu.sync_copy(buf1, acc_sh.at[dst_v.at[r1]], add=True)
        return carry

      lax.fori_loop(0, G // 2, body, 0)

    plsc.subcore_barrier()
    pltpu.sync_copy(acc_sh.at[pl.ds(base, RPT)],
                    out_hbm.at[c, pl.ds(base, RPT)])

  return seg


_segsum_l1 = _make_segsum(NR_E, edge_split=True)
_segsum_hid = _make_segsum(NR_F, edge_split=False)


def _bn_body(x_ref, sc_ref, sh_ref, out_ref):
  out_ref[...] = x_ref[...] * sc_ref[...] + sh_ref[...]


def _layer1_body(h_ref, agg_ref, w_ref, b_ref, eps_ref, out_ref):
  e = 1.0 + eps_ref[0, 0]
  u = e * h_ref[...] + agg_ref[0, 0:N, :] + agg_ref[1, 0:N, :]
  z = jnp.dot(u, w_ref[...], preferred_element_type=jnp.float32) + b_ref[...]
  z = jnp.maximum(z, 0.0)
  out_ref[0:N] = z[:, 0:128]
  out_ref[N:2 * N] = z[:, 128:256]


def _layer2_body(h_ref, agg_ref, w_ref, b_ref, eps_ref, out_ref):
  e = 1.0 + eps_ref[0, 0]
  u = e * h_ref[0:N, :] + agg_ref[0, 0:N, :]
  v = e * h_ref[N:2 * N, :] + agg_ref[1, 0:N, :]
  z = (jnp.dot(u, w_ref[0:128, :], preferred_element_type=jnp.float32)
       + jnp.dot(v, w_ref[128:256, :], preferred_element_type=jnp.float32)
       + b_ref[...])
  z = jnp.maximum(z, 0.0)
  out_ref[0:N] = z[:, 0:128]
  out_ref[N:2 * N] = z[:, 128:256]


def _layer3_body(h_ref, agg_ref, w3_ref, b3_ref, eps_ref, w4_ref, b4_ref,
                 out_ref):
  e = 1.0 + eps_ref[0, 0]
  u = e * h_ref[0:N, :] + agg_ref[0, 0:N, :]
  v = e * h_ref[N:2 * N, :] + agg_ref[1, 0:N, :]
  t = (jnp.dot(u, w3_ref[0:128, :], preferred_element_type=jnp.float32)
       + jnp.dot(v, w3_ref[128:256, :], preferred_element_type=jnp.float32)
       + b3_ref[...])
  t = jnp.maximum(t, 0.0)
  out_ref[...] = (jnp.dot(t, w4_ref[...], preferred_element_type=jnp.float32)
                  + b4_ref[...])


def _tc_call(body, out_shape, smem_arg_idxs, *args):
  specs = [pl.BlockSpec(memory_space=pltpu.VMEM) for _ in args]
  for i in smem_arg_idxs:
    specs[i] = pl.BlockSpec(memory_space=pltpu.SMEM)
  return pl.pallas_call(
      body,
      out_shape=jax.ShapeDtypeStruct(out_shape, jnp.float32),
      in_specs=specs,
      out_specs=pl.BlockSpec(memory_space=pltpu.VMEM),
  )(*args)


def kernel(x, edge_index, batch, bn_gamma, bn_beta,
           W1, b1, eps1, W2, b2, eps2, W3, b3, eps3, W4, b4):
  del batch
  f32 = jnp.float32
  scale = (bn_gamma * np.float32(1.0 / np.sqrt(1.0 + 1e-5))).reshape(1, D)
  shift = bn_beta.reshape(1, D)

  src = edge_index[0]
  dst = edge_index[1]
  pad = EPAD - E
  srcp = jnp.concatenate([src, jnp.zeros((pad,), jnp.int32)])
  dummy = N + (jnp.arange(pad, dtype=jnp.int32) % (NACC - N))
  dstp = jnp.concatenate([dst, dummy])
  src_e = srcp.reshape(2, NSUB, NR_E, CHUNK)
  dst_e = dstp.reshape(2, NSUB, NR_E, CHUNK)
  src_f = jnp.stack([srcp, srcp + N]).reshape(2, NSUB, NR_F, CHUNK)
  dst_f = dstp.reshape(NSUB, NR_F, CHUNK)
  zeros = jnp.zeros((NACC, 128), f32)

  e1 = eps1.astype(f32).reshape(1, 1)
  e2 = eps2.astype(f32).reshape(1, 1)
  e3 = eps3.astype(f32).reshape(1, 1)

  h0 = _tc_call(_bn_body, (N, D), (), x, scale, shift)
  agg1 = _segsum_l1(h0, src_e, dst_e, zeros)
  h1 = _tc_call(_layer1_body, (2 * N, 128), (4,),
                h0, agg1, W1, b1.reshape(1, HID), e1)
  agg2 = _segsum_hid(h1, src_f, dst_f, zeros)
  h2 = _tc_call(_layer2_body, (2 * N, 128), (4,),
                h1, agg2, W2, b2.reshape(1, HID), e2)
  agg3 = _segsum_hid(h2, src_f, dst_f, zeros)
  out = _tc_call(_layer3_body, (N, OUT), (4,), h2, agg3, W3,
                 b3.reshape(1, HID), e3, W4, b4.reshape(1, OUT))
  return out


# speedup vs baseline: 13.5799x; 4.4064x over previous
import functools

import jax
import jax.numpy as jnp
import numpy as np
from jax import lax
from jax.experimental import pallas as pl
from jax.experimental.pallas import tpu as pltpu
from jax.experimental.pallas import tpu_sc as plsc

N = 10000
E = 320000
D = 128
HID = 256
OUT = 128

NSUB = 16
CHUNK = 128
EPAD = 327680
NR_E = 80
NR_F = 160
G = 40
NACC = 10112
RPT = NACC // NSUB


def _make_segsum(nrows, edge_split):
  mesh = plsc.VectorSubcoreMesh(core_axis_name="c", subcore_axis_name="s")
  ngroups = nrows // G

  @functools.partial(
      pl.kernel,
      out_type=jax.ShapeDtypeStruct((2, NACC, 128), jnp.float32),
      mesh=mesh,
      scratch_types=[
          pltpu.VMEM((G, CHUNK), jnp.int32),
          pltpu.VMEM((G, CHUNK), jnp.int32),
          pltpu.VMEM((CHUNK, 128), jnp.float32),
          pltpu.VMEM((CHUNK, 128), jnp.float32),
          pltpu.VMEM_SHARED((NACC, 128), jnp.float32),
          pltpu.SemaphoreType.DMA,
          pltpu.SemaphoreType.DMA,
      ],
  )
  def seg(h_hbm, src_hbm, dst_hbm, zeros_hbm, out_hbm,
          src_v, dst_v, buf0, buf1, acc_sh, sem0, sem1):
    c = lax.axis_index("c")
    s = lax.axis_index("s")
    base = s * RPT
    pltpu.sync_copy(zeros_hbm.at[pl.ds(base, RPT)],
                    acc_sh.at[pl.ds(base, RPT)])
    plsc.subcore_barrier()

    for g in range(ngroups):
      pltpu.sync_copy(src_hbm.at[c, s, pl.ds(g * G, G)], src_v)
      if edge_split:
        pltpu.sync_copy(dst_hbm.at[c, s, pl.ds(g * G, G)], dst_v)
      else:
        pltpu.sync_copy(dst_hbm.at[s, pl.ds(g * G, G)], dst_v)

      import os as _os
      _mode = _os.environ.get("SEGMODE", "full")

      def body(j2, carry):
        r0 = 2 * j2
        r1 = r0 + 1
        if _mode in ("full", "gather"):
          cp0 = pltpu.async_copy(h_hbm.at[src_v.at[r0]], buf0, sem0)
          cp1 = pltpu.async_copy(h_hbm.at[src_v.at[r1]], buf1, sem1)
          cp0.wait()
          cp1.wait()
        if _mode in ("full", "scatter"):
          pltpu.sync_copy(buf0, acc_sh.at[dst_v.at[r0]], add=True)
          pltpu.sync_copy(buf1, acc_sh.at[dst_v.at[r1]], add=True)
        return carry

      lax.fori_loop(0, G // 2, body, 0)

    plsc.subcore_barrier()
    pltpu.sync_copy(acc_sh.at[pl.ds(base, RPT)],
                    out_hbm.at[c, pl.ds(base, RPT)])

  return seg


_segsum_l1 = _make_segsum(NR_E, edge_split=True)
_segsum_hid = _make_segsum(NR_F, edge_split=False)


def _bn_body(x_ref, sc_ref, sh_ref, out_ref):
  out_ref[...] = x_ref[...] * sc_ref[...] + sh_ref[...]


def _layer1_body(h_ref, agg_ref, w_ref, b_ref, eps_ref, out_ref):
  e = 1.0 + eps_ref[0, 0]
  u = e * h_ref[...] + agg_ref[0, 0:N, :] + agg_ref[1, 0:N, :]
  z = jnp.dot(u, w_ref[...], preferred_element_type=jnp.float32) + b_ref[...]
  z = jnp.maximum(z, 0.0)
  out_ref[0:N] = z[:, 0:128]
  out_ref[N:2 * N] = z[:, 128:256]


def _layer2_body(h_ref, agg_ref, w_ref, b_ref, eps_ref, out_ref):
  e = 1.0 + eps_ref[0, 0]
  u = e * h_ref[0:N, :] + agg_ref[0, 0:N, :]
  v = e * h_ref[N:2 * N, :] + agg_ref[1, 0:N, :]
  z = (jnp.dot(u, w_ref[0:128, :], preferred_element_type=jnp.float32)
       + jnp.dot(v, w_ref[128:256, :], preferred_element_type=jnp.float32)
       + b_ref[...])
  z = jnp.maximum(z, 0.0)
  out_ref[0:N] = z[:, 0:128]
  out_ref[N:2 * N] = z[:, 128:256]


def _layer3_body(h_ref, agg_ref, w3_ref, b3_ref, eps_ref, w4_ref, b4_ref,
                 out_ref):
  e = 1.0 + eps_ref[0, 0]
  u = e * h_ref[0:N, :] + agg_ref[0, 0:N, :]
  v = e * h_ref[N:2 * N, :] + agg_ref[1, 0:N, :]
  t = (jnp.dot(u, w3_ref[0:128, :], preferred_element_type=jnp.float32)
       + jnp.dot(v, w3_ref[128:256, :], preferred_element_type=jnp.float32)
       + b3_ref[...])
  t = jnp.maximum(t, 0.0)
  out_ref[...] = (jnp.dot(t, w4_ref[...], preferred_element_type=jnp.float32)
                  + b4_ref[...])


def _tc_call(body, out_shape, smem_arg_idxs, *args):
  specs = [pl.BlockSpec(memory_space=pltpu.VMEM) for _ in args]
  for i in smem_arg_idxs:
    specs[i] = pl.BlockSpec(memory_space=pltpu.SMEM)
  return pl.pallas_call(
      body,
      out_shape=jax.ShapeDtypeStruct(out_shape, jnp.float32),
      in_specs=specs,
      out_specs=pl.BlockSpec(memory_space=pltpu.VMEM),
  )(*args)


def kernel(x, edge_index, batch, bn_gamma, bn_beta,
           W1, b1, eps1, W2, b2, eps2, W3, b3, eps3, W4, b4):
  del batch
  f32 = jnp.float32
  scale = (bn_gamma * np.float32(1.0 / np.sqrt(1.0 + 1e-5))).reshape(1, D)
  shift = bn_beta.reshape(1, D)

  src = edge_index[0]
  dst = edge_index[1]
  pad = EPAD - E
  srcp = jnp.concatenate([src, jnp.zeros((pad,), jnp.int32)])
  dummy = N + (jnp.arange(pad, dtype=jnp.int32) % (NACC - N))
  dstp = jnp.concatenate([dst, dummy])
  src_e = srcp.reshape(2, NSUB, NR_E, CHUNK)
  dst_e = dstp.reshape(2, NSUB, NR_E, CHUNK)
  src_f = jnp.stack([srcp, srcp + N]).reshape(2, NSUB, NR_F, CHUNK)
  dst_f = dstp.reshape(NSUB, NR_F, CHUNK)
  zeros = jnp.zeros((NACC, 128), f32)

  e1 = eps1.astype(f32).reshape(1, 1)
  e2 = eps2.astype(f32).reshape(1, 1)
  e3 = eps3.astype(f32).reshape(1, 1)

  h0 = _tc_call(_bn_body, (N, D), (), x, scale, shift)
  agg1 = _segsum_l1(h0, src_e, dst_e, zeros)
  h1 = _tc_call(_layer1_body, (2 * N, 128), (4,),
                h0, agg1, W1, b1.reshape(1, HID), e1)
  agg2 = _segsum_hid(h1, src_f, dst_f, zeros)
  h2 = _tc_call(_layer2_body, (2 * N, 128), (4,),
                h1, agg2, W2, b2.reshape(1, HID), e2)
  agg3 = _segsum_hid(h2, src_f, dst_f, zeros)
  out = _tc_call(_layer3_body, (N, OUT), (4,), h2, agg3, W3,
                 b3.reshape(1, HID), e3, W4, b4.reshape(1, OUT))
  return out
